# baseline (speedup 1.0000x reference)
import functools

import jax
import jax.numpy as jnp
from jax import lax
from jax.experimental import pallas as pl
from jax.experimental.pallas import tpu as pltpu
from jax.experimental.pallas import tpu_sc as plsc

N = 10000
K_NEIGH = 32
D_FEAT = 128
IN_DIM = 2 * D_FEAT
OUT_DIM = 128

BLOCK_N = 496
TC_BLOCKS = 15
TC_ROWS = TC_BLOCKS * BLOCK_N
SC_ROWS = N - TC_ROWS

NUM_CORES = 2
NUM_SUBCORES = 16
NUM_WORKERS = NUM_CORES * NUM_SUBCORES
ROWS_PER_WORKER = SC_ROWS // NUM_WORKERS
ROWS_PER_CORE = SC_ROWS // NUM_CORES

SCATTER_ROWS = 4
CHUNK_ROWS = 8
NUM_CHUNKS = ROWS_PER_WORKER // CHUNK_ROWS


def _tc_main_body(feat_ref, emb_ref, w_ref, b_ref, out_ref):
    emb = emb_ref[...]
    m = jnp.mean(emb, axis=1)
    x = jnp.concatenate([feat_ref[...], m], axis=1)
    y = jnp.dot(x, w_ref[...], preferred_element_type=jnp.float32)
    out_ref[...] = jnp.maximum(y + b_ref[...], 0.0)


def _tc_tail_body(feat_hbm, agg_ref, w_ref, b_ref, out_ref, feat_v, sem):
    cp = pltpu.make_async_copy(
        feat_hbm.at[pl.ds(TC_ROWS, SC_ROWS), :], feat_v, sem)
    cp.start()
    cp.wait()
    m = agg_ref[...] * (1.0 / K_NEIGH)
    x = jnp.concatenate([feat_v[...], m], axis=1)
    y = jnp.dot(x, w_ref[...], preferred_element_type=jnp.float32)
    out_ref[...] = jnp.maximum(y + b_ref[...], 0.0)


def _sc_neighbor_sum(emb2d, zeros_hbm, out_hbm, buf0, buf1, idx_v, acc_sh,
                     rsem0, rsem1):
    c = lax.axis_index("c")
    s = lax.axis_index("s")
    local_base = s * ROWS_PER_WORKER
    glob_base = c * ROWS_PER_CORE + s * ROWS_PER_WORKER
    src_base = (TC_ROWS + glob_base) * K_NEIGH

    pltpu.sync_copy(
        zeros_hbm.at[pl.ds(local_base, ROWS_PER_WORKER)],
        acc_sh.at[pl.ds(local_base, ROWS_PER_WORKER)],
    )

    iota = lax.broadcasted_iota(jnp.int32, (16,), 0)
    for j16 in range(128 // 16):
        vals = lax.shift_right_logical(iota + (16 * j16), 5) + local_base
        idx_v[pl.ds(j16 * 16, 16)] = vals

    bufs = (buf0, buf1)
    sems = (rsem0, rsem1)
    reads = [None] * NUM_CHUNKS

    def _issue_read(chunk_id):
        base = src_base + chunk_id * CHUNK_ROWS * K_NEIGH
        reads[chunk_id] = pltpu.async_copy(
            emb2d.at[pl.ds(base, CHUNK_ROWS * K_NEIGH), :],
            bufs[chunk_id % 2],
            sems[chunk_id % 2],
        )

    _issue_read(0)
    for cidx in range(NUM_CHUNKS):
        if cidx + 1 < NUM_CHUNKS:
            _issue_read(cidx + 1)
        buf = bufs[cidx % 2]
        reads[cidx].wait()
        for b in range(CHUNK_ROWS // SCATTER_ROWS):
            pltpu.sync_copy(
                buf.at[pl.ds(b * SCATTER_ROWS * K_NEIGH,
                             SCATTER_ROWS * K_NEIGH)],
                acc_sh.at[idx_v],
                add=True,
            )
            for j16 in range(128 // 16):
                sl = pl.ds(j16 * 16, 16)
                idx_v[sl] = idx_v[sl] + SCATTER_ROWS

    pltpu.sync_copy(
        acc_sh.at[pl.ds(local_base, ROWS_PER_WORKER)],
        out_hbm.at[pl.ds(glob_base, ROWS_PER_WORKER)],
    )


_sc_sum_call = functools.partial(
    pl.kernel,
    mesh=plsc.VectorSubcoreMesh(core_axis_name="c", subcore_axis_name="s"),
    out_type=jax.ShapeDtypeStruct((SC_ROWS, D_FEAT), jnp.float32),
    scratch_types=[
        pltpu.VMEM((CHUNK_ROWS * K_NEIGH, D_FEAT), jnp.float32),
        pltpu.VMEM((CHUNK_ROWS * K_NEIGH, D_FEAT), jnp.float32),
        pltpu.VMEM((128,), jnp.int32),
        pltpu.VMEM_SHARED((ROWS_PER_CORE, D_FEAT), jnp.float32),
        pltpu.SemaphoreType.DMA,
        pltpu.SemaphoreType.DMA,
    ],
)(_sc_neighbor_sum)


def kernel(features, embedding_look_up, kernel, bias_weights):
    bias2d = bias_weights.reshape(1, OUT_DIM)
    emb2d = embedding_look_up.reshape(N * K_NEIGH, D_FEAT)
    zeros_hbm = jnp.zeros((ROWS_PER_CORE, D_FEAT), jnp.float32)

    agg_sum = _sc_sum_call(emb2d, zeros_hbm)

    out_main = pl.pallas_call(
        _tc_main_body,
        grid=(TC_BLOCKS,),
        in_specs=[
            pl.BlockSpec((BLOCK_N, D_FEAT), lambda i: (i, 0)),
            pl.BlockSpec((BLOCK_N, K_NEIGH, D_FEAT), lambda i: (i, 0, 0)),
            pl.BlockSpec((IN_DIM, OUT_DIM), lambda i: (0, 0)),
            pl.BlockSpec((1, OUT_DIM), lambda i: (0, 0)),
        ],
        out_specs=pl.BlockSpec((BLOCK_N, OUT_DIM), lambda i: (i, 0)),
        out_shape=jax.ShapeDtypeStruct((TC_ROWS, OUT_DIM), jnp.float32),
        compiler_params=pltpu.CompilerParams(
            dimension_semantics=("parallel",),
        ),
    )(features, embedding_look_up, kernel, bias2d)

    out_tail = pl.pallas_call(
        _tc_tail_body,
        grid=(1,),
        in_specs=[
            pl.BlockSpec(memory_space=pltpu.MemorySpace.HBM),
            pl.BlockSpec((SC_ROWS, D_FEAT), lambda i: (0, 0)),
            pl.BlockSpec((IN_DIM, OUT_DIM), lambda i: (0, 0)),
            pl.BlockSpec((1, OUT_DIM), lambda i: (0, 0)),
        ],
        out_specs=pl.BlockSpec((SC_ROWS, OUT_DIM), lambda i: (0, 0)),
        out_shape=jax.ShapeDtypeStruct((SC_ROWS, OUT_DIM), jnp.float32),
        scratch_shapes=[
            pltpu.VMEM((SC_ROWS, D_FEAT), jnp.float32),
            pltpu.SemaphoreType.DMA,
        ],
    )(features, agg_sum, kernel, bias2d)

    return jnp.concatenate([out_main, out_tail], axis=0)


# speedup vs baseline: 1.4713x; 1.4713x over previous
import jax
import jax.numpy as jnp
from jax.experimental import pallas as pl
from jax.experimental.pallas import tpu as pltpu

N = 10000
K_NEIGH = 32
D_FEAT = 128
IN_DIM = 2 * D_FEAT
OUT_DIM = 128

BLOCK_N = 400


def _agg_body(feat_ref, emb_ref, w_ref, b_ref, out_ref):
    emb = emb_ref[...]
    m = jnp.mean(emb, axis=1)
    x = jnp.concatenate([feat_ref[...], m], axis=1)
    y = jnp.dot(x, w_ref[...], preferred_element_type=jnp.float32)
    out_ref[...] = jnp.maximum(y + b_ref[...], 0.0)


def kernel(features, embedding_look_up, kernel, bias_weights):
    bias2d = bias_weights.reshape(1, OUT_DIM)
    return pl.pallas_call(
        _agg_body,
        grid=(N // BLOCK_N,),
        in_specs=[
            pl.BlockSpec((BLOCK_N, D_FEAT), lambda i: (i, 0)),
            pl.BlockSpec((BLOCK_N, K_NEIGH, D_FEAT), lambda i: (i, 0, 0)),
            pl.BlockSpec((IN_DIM, OUT_DIM), lambda i: (0, 0)),
            pl.BlockSpec((1, OUT_DIM), lambda i: (0, 0)),
        ],
        out_specs=pl.BlockSpec((BLOCK_N, OUT_DIM), lambda i: (i, 0)),
        out_shape=jax.ShapeDtypeStruct((N, OUT_DIM), jnp.float32),
        compiler_params=pltpu.CompilerParams(
            dimension_semantics=("parallel",),
        ),
    )(features, embedding_look_up, kernel, bias2d)
